# baseline (speedup 1.0000x reference)
import functools

import jax
import jax.numpy as jnp
from jax import lax
from jax.experimental import pallas as pl
from jax.experimental.pallas import tpu as pltpu
from jax.experimental.pallas import tpu_sc as plsc

N = 10000
E = 320000
D = 128
NPAD = 10240
NC, NS = 2, 16
NW = NC * NS
B = 80
CHUNKS = E // B
MINCH = CHUNKS // NW
L = 16
ROWS_PER_SUB = NPAD // NS


def _mm_body(x_ref, wl_ref, wr_ref, xl_ref, xr_ref):
    xb = x_ref[...]
    xl_ref[...] = jnp.dot(xb, wl_ref[...], preferred_element_type=jnp.float32)
    xr_ref[...] = jnp.dot(xb, wr_ref[...], preferred_element_type=jnp.float32)


def _matmuls(x, W_l, W_r):
    blk = 2000
    grid = N // blk
    return pl.pallas_call(
        _mm_body,
        grid=(grid,),
        in_specs=[
            pl.BlockSpec((blk, D), lambda i: (i, 0)),
            pl.BlockSpec((D, D), lambda i: (0, 0)),
            pl.BlockSpec((D, D), lambda i: (0, 0)),
        ],
        out_specs=[
            pl.BlockSpec((blk, D), lambda i: (i, 0)),
            pl.BlockSpec((blk, D), lambda i: (i, 0)),
        ],
        out_shape=[
            jax.ShapeDtypeStruct((N, D), jnp.float32),
            jax.ShapeDtypeStruct((N, D), jnp.float32),
        ],
    )(x, W_l, W_r)


_sc_mesh = plsc.VectorSubcoreMesh(core_axis_name="c", subcore_axis_name="s")


@functools.partial(
    pl.kernel,
    out_type=(
        jax.ShapeDtypeStruct((NC, NPAD, D), jnp.float32),
        jax.ShapeDtypeStruct((NC, NPAD), jnp.float32),
    ),
    mesh=_sc_mesh,
    compiler_params=pltpu.CompilerParams(needs_layout_passes=False),
    scratch_types=dict(
        acc_sh=pltpu.VMEM_SHARED((NPAD, D), jnp.float32),
        s_sh=pltpu.VMEM_SHARED((NPAD,), jnp.float32),
        xl_rows0=pltpu.VMEM((B, D), jnp.float32),
        xr_rows0=pltpu.VMEM((B, D), jnp.float32),
        xl_rows1=pltpu.VMEM((B, D), jnp.float32),
        xr_rows1=pltpu.VMEM((B, D), jnp.float32),
        src_i0=pltpu.VMEM((B,), jnp.int32),
        dst_i0=pltpu.VMEM((B,), jnp.int32),
        src_i1=pltpu.VMEM((B,), jnp.int32),
        dst_i1=pltpu.VMEM((B,), jnp.int32),
        src_i2=pltpu.VMEM((B,), jnp.int32),
        dst_i2=pltpu.VMEM((B,), jnp.int32),
        src_i3=pltpu.VMEM((B,), jnp.int32),
        dst_i3=pltpu.VMEM((B,), jnp.int32),
        e_stage=pltpu.VMEM((B * L,), jnp.float32),
        w_buf0=pltpu.VMEM((B,), jnp.float32),
        w_buf1=pltpu.VMEM((B,), jnp.float32),
        att_v=pltpu.VMEM((D,), jnp.float32),
        gsem0=pltpu.SemaphoreType.DMA,
        gsem1=pltpu.SemaphoreType.DMA,
        ssem0=pltpu.SemaphoreType.DMA,
        ssem1=pltpu.SemaphoreType.DMA,
        isem0=pltpu.SemaphoreType.DMA,
        isem1=pltpu.SemaphoreType.DMA,
        isem2=pltpu.SemaphoreType.DMA,
        isem3=pltpu.SemaphoreType.DMA,
    ),
)
def _edge_pass(xl_hbm, xr_hbm, src_hbm, dst_hbm, att_hbm,
               acc_out, s_out,
               acc_sh, s_sh, xl_rows0, xr_rows0, xl_rows1, xr_rows1,
               src_i0, dst_i0, src_i1, dst_i1, src_i2, dst_i2, src_i3, dst_i3,
               e_stage, w_buf0, w_buf1, att_v,
               gsem0, gsem1, ssem0, ssem1, isem0, isem1, isem2, isem3):
    cid = lax.axis_index("c")
    sid = lax.axis_index("s")
    wid = sid * NC + cid
    rbufs = ((xl_rows0, xr_rows0, w_buf0, gsem0, ssem0),
             (xl_rows1, xr_rows1, w_buf1, gsem1, ssem1))
    ibufs = ((src_i0, dst_i0, isem0), (src_i1, dst_i1, isem1),
             (src_i2, dst_i2, isem2), (src_i3, dst_i3, isem3))

    pltpu.sync_copy(att_hbm, att_v)

    z16 = jnp.zeros((L,), jnp.float32)

    def _zrow(j, _):
        for k in range(D // L):
            xl_rows0[j, pl.ds(k * L, L)] = z16
        return 0

    lax.fori_loop(0, B, _zrow, 0)

    for g in range(B // L):
        w_buf0[pl.ds(g * L, L)] = z16
    row0 = sid * ROWS_PER_SUB
    for j in range(ROWS_PER_SUB // B):
        pltpu.sync_copy(xl_rows0, acc_sh.at[pl.ds(row0 + j * B, B)])
        pltpu.sync_copy(w_buf0, s_sh.at[pl.ds(row0 + j * B, B)])
    plsc.subcore_barrier()

    att_regs = [att_v[pl.ds(k * L, L)] for k in range(D // L)]

    def _issue_idx(t, ii):
        src_i, dst_i, isem = ibufs[ii]
        base = pl.multiple_of((wid + t * NW) * B, B)
        pltpu.make_async_copy(src_hbm.at[pl.ds(base, B)], src_i, isem).start()
        pltpu.make_async_copy(dst_hbm.at[pl.ds(base, B)], dst_i, isem).start()

    def _wait_idx(ii):
        src_i, dst_i, isem = ibufs[ii]
        pltpu.make_async_copy(src_hbm.at[pl.ds(0, B)], src_i, isem).wait()
        pltpu.make_async_copy(dst_hbm.at[pl.ds(0, B)], dst_i, isem).wait()

    def _wait_scatters(rb):
        xl_rows, xr_rows, w_buf, gsem, ssem = rbufs[rb]
        pltpu.make_async_copy(xl_rows, acc_sh.at[dst_i0], ssem).wait()
        pltpu.make_async_copy(w_buf, s_sh.at[dst_i0], ssem).wait()

    def _start_xr(rb, ii):
        xl_rows, xr_rows, w_buf, gsem, ssem = rbufs[rb]
        src_i, dst_i, isem = ibufs[ii]
        pltpu.make_async_copy(xr_hbm.at[dst_i], xr_rows, gsem).start()

    def _start_xl(rb, ii):
        xl_rows, xr_rows, w_buf, gsem, ssem = rbufs[rb]
        src_i, dst_i, isem = ibufs[ii]
        pltpu.make_async_copy(xl_hbm.at[src_i], xl_rows, gsem).start()

    def _section(t, j):
        p, q = j % 2, (j + 1) % 2
        xl_rows, xr_rows, w_buf, gsem, ssem = rbufs[p]
        src_i, dst_i, isem = ibufs[j]
        pltpu.make_async_copy(xl_hbm.at[src_i], xl_rows, gsem).wait()
        pltpu.make_async_copy(xr_hbm.at[dst_i], xr_rows, gsem).wait()

        def _edge_e(i, _):
            accs = [jnp.zeros((L,), jnp.float32), jnp.zeros((L,), jnp.float32)]
            for k in range(D // L):
                h = xl_rows[i, pl.ds(k * L, L)] + xr_rows[i, pl.ds(k * L, L)]
                h = jnp.maximum(h, h * 0.2)
                accs[k % 2] = accs[k % 2] + h * att_regs[k]
            e_stage[pl.ds(i * L, L)] = accs[0] + accs[1]
            return 0

        lax.fori_loop(0, B, _edge_e, 0, unroll=4)

        @pl.when(t + 2 < MINCH)
        def _():
            _wait_idx((j + 2) % 4)
            _start_xr(p, (j + 2) % 4)

        @pl.when(t >= 1)
        def _():
            _wait_scatters(q)

        @pl.when(t + 1 < MINCH)
        def _():
            _start_xl(q, (j + 1) % 4)

        lane = lax.iota(jnp.int32, L)
        for g in range(B // L):
            fbase = lane * L + (g * L * L)
            ev = jnp.zeros((L,), jnp.float32)
            for k in range(L):
                ev = ev + plsc.load_gather(e_stage, [fbase + k])
            w_buf[pl.ds(g * L, L)] = jnp.exp(ev)

        def _edge_scale(i, _):
            wj = plsc.load_gather(w_buf, [jnp.full((L,), i, jnp.int32)])
            for k in range(D // L):
                sl = pl.ds(k * L, L)
                xl_rows[i, sl] = xl_rows[i, sl] * wj
            return 0

        lax.fori_loop(0, B, _edge_scale, 0, unroll=4)

        pltpu.async_copy(xl_rows, acc_sh.at[dst_i], ssem, add=True)
        pltpu.async_copy(w_buf, s_sh.at[dst_i], ssem, add=True)

        @pl.when(t + 3 < MINCH)
        def _():
            _issue_idx(t + 3, (j + 3) % 4)

    _issue_idx(0, 0)
    _issue_idx(1, 1)
    _issue_idx(2, 2)
    _wait_idx(0)
    _start_xl(0, 0)
    _start_xr(0, 0)
    _wait_idx(1)
    _start_xr(1, 1)

    def _quad(qq, _):
        for j in range(4):
            _section(4 * qq + j, j)
        return 0

    lax.fori_loop(0, MINCH // 4, _quad, 0)
    if MINCH % 4 == 1:
        _section(MINCH - 1, 0)
    else:
        raise NotImplementedError

    _wait_scatters(0)
    plsc.subcore_barrier()
    pltpu.sync_copy(acc_sh.at[pl.ds(row0, ROWS_PER_SUB)],
                    acc_out.at[cid, pl.ds(row0, ROWS_PER_SUB)])
    pltpu.sync_copy(s_sh.at[pl.ds(row0, ROWS_PER_SUB)],
                    s_out.at[cid, pl.ds(row0, ROWS_PER_SUB)])


def _fin_body(xl_ref, xr_ref, acc_ref, s_ref, att_ref, bias_ref, gamma_ref,
              beta_ref, out_ref):
    xl = xl_ref[...]
    xr = xr_ref[...]
    att = att_ref[...]
    h = xl + xr
    h = jnp.maximum(h, h * 0.2)
    e_self = jnp.sum(h * att, axis=1, keepdims=True)
    w_self = jnp.exp(e_self)
    num = acc_ref[0] + acc_ref[1] + w_self * xl
    den = jnp.sum(s_ref[...], axis=1, keepdims=True) + w_self
    out = num / den + bias_ref[...]
    out = jnp.maximum(out, out * 0.01)
    mu = jnp.mean(out, axis=1, keepdims=True)
    c = out - mu
    var = jnp.mean(c * c, axis=1, keepdims=True)
    out_ref[...] = c * lax.rsqrt(var + 1e-5) * gamma_ref[...] + beta_ref[...]


def _finish(xl, xr, acc2, s2t, att, bias, gamma, beta):
    blk = 2000
    grid = N // blk
    return pl.pallas_call(
        _fin_body,
        grid=(grid,),
        in_specs=[
            pl.BlockSpec((blk, D), lambda i: (i, 0)),
            pl.BlockSpec((blk, D), lambda i: (i, 0)),
            pl.BlockSpec((NC, blk, D), lambda i: (0, i, 0)),
            pl.BlockSpec((blk, NC), lambda i: (i, 0)),
            pl.BlockSpec((1, D), lambda i: (0, 0)),
            pl.BlockSpec((1, D), lambda i: (0, 0)),
            pl.BlockSpec((1, D), lambda i: (0, 0)),
            pl.BlockSpec((1, D), lambda i: (0, 0)),
        ],
        out_specs=pl.BlockSpec((blk, D), lambda i: (i, 0)),
        out_shape=jax.ShapeDtypeStruct((N, D), jnp.float32),
    )(xl, xr, acc2, s2t, att, bias, gamma, beta)


def kernel(x, edge_index, W_l, W_r, att, bias, gamma, beta):
    xl, xr = _matmuls(x, W_l, W_r)
    acc2, s2 = _edge_pass(xl, xr, edge_index[0], edge_index[1], att)
    s2t = s2.T
    return _finish(xl, xr, acc2, s2t,
                   att.reshape(1, D), bias.reshape(1, D),
                   gamma.reshape(1, D), beta.reshape(1, D))


# speedup vs baseline: 1.0887x; 1.0887x over previous
import functools

import jax
import jax.numpy as jnp
from jax import lax
from jax.experimental import pallas as pl
from jax.experimental.pallas import tpu as pltpu
from jax.experimental.pallas import tpu_sc as plsc

N = 10000
E = 320000
D = 128
NPAD = 10240
NC, NS = 2, 16
NW = NC * NS
B = 80
CHUNKS = E // B
MINCH = CHUNKS // NW
L = 16
ROWS_PER_SUB = NPAD // NS


def _mm_body(x_ref, wl_ref, wr_ref, xl_ref, xr_ref):
    xb = x_ref[...]
    xl_ref[...] = jnp.dot(xb, wl_ref[...], preferred_element_type=jnp.float32)
    xr_ref[...] = jnp.dot(xb, wr_ref[...], preferred_element_type=jnp.float32)


def _matmuls(x, W_l, W_r):
    blk = 2000
    grid = N // blk
    return pl.pallas_call(
        _mm_body,
        grid=(grid,),
        in_specs=[
            pl.BlockSpec((blk, D), lambda i: (i, 0)),
            pl.BlockSpec((D, D), lambda i: (0, 0)),
            pl.BlockSpec((D, D), lambda i: (0, 0)),
        ],
        out_specs=[
            pl.BlockSpec((blk, D), lambda i: (i, 0)),
            pl.BlockSpec((blk, D), lambda i: (i, 0)),
        ],
        out_shape=[
            jax.ShapeDtypeStruct((N, D), jnp.float32),
            jax.ShapeDtypeStruct((N, D), jnp.float32),
        ],
    )(x, W_l, W_r)


_sc_mesh = plsc.VectorSubcoreMesh(core_axis_name="c", subcore_axis_name="s")


@functools.partial(
    pl.kernel,
    out_type=(
        jax.ShapeDtypeStruct((NC, NPAD, D), jnp.float32),
        jax.ShapeDtypeStruct((NC, NPAD), jnp.float32),
    ),
    mesh=_sc_mesh,
    compiler_params=pltpu.CompilerParams(needs_layout_passes=False),
    scratch_types=dict(
        acc_sh=pltpu.VMEM_SHARED((NPAD, D), jnp.float32),
        s_sh=pltpu.VMEM_SHARED((NPAD,), jnp.float32),
        xl_rows0=pltpu.VMEM((B, D), jnp.float32),
        xr_rows0=pltpu.VMEM((B, D), jnp.float32),
        xl_rows1=pltpu.VMEM((B, D), jnp.float32),
        xr_rows1=pltpu.VMEM((B, D), jnp.float32),
        src_i0=pltpu.VMEM((B,), jnp.int32),
        dst_i0=pltpu.VMEM((B,), jnp.int32),
        src_i1=pltpu.VMEM((B,), jnp.int32),
        dst_i1=pltpu.VMEM((B,), jnp.int32),
        src_i2=pltpu.VMEM((B,), jnp.int32),
        dst_i2=pltpu.VMEM((B,), jnp.int32),
        src_i3=pltpu.VMEM((B,), jnp.int32),
        dst_i3=pltpu.VMEM((B,), jnp.int32),
        e_stage=pltpu.VMEM((B * L,), jnp.float32),
        w_buf0=pltpu.VMEM((B,), jnp.float32),
        w_buf1=pltpu.VMEM((B,), jnp.float32),
        att_v=pltpu.VMEM((D,), jnp.float32),
        gsem0=pltpu.SemaphoreType.DMA,
        gsem1=pltpu.SemaphoreType.DMA,
        ssem0=pltpu.SemaphoreType.DMA,
        ssem1=pltpu.SemaphoreType.DMA,
        isem0=pltpu.SemaphoreType.DMA,
        isem1=pltpu.SemaphoreType.DMA,
        isem2=pltpu.SemaphoreType.DMA,
        isem3=pltpu.SemaphoreType.DMA,
    ),
)
def _edge_pass(xl_hbm, xr_hbm, src_hbm, dst_hbm, att_hbm,
               acc_out, s_out,
               acc_sh, s_sh, xl_rows0, xr_rows0, xl_rows1, xr_rows1,
               src_i0, dst_i0, src_i1, dst_i1, src_i2, dst_i2, src_i3, dst_i3,
               e_stage, w_buf0, w_buf1, att_v,
               gsem0, gsem1, ssem0, ssem1, isem0, isem1, isem2, isem3):
    cid = lax.axis_index("c")
    sid = lax.axis_index("s")
    wid = sid * NC + cid
    rbufs = ((xl_rows0, xr_rows0, w_buf0, gsem0, ssem0),
             (xl_rows1, xr_rows1, w_buf1, gsem1, ssem1))
    ibufs = ((src_i0, dst_i0, isem0), (src_i1, dst_i1, isem1),
             (src_i2, dst_i2, isem2), (src_i3, dst_i3, isem3))

    pltpu.sync_copy(att_hbm, att_v)

    z16 = jnp.zeros((L,), jnp.float32)

    def _zrow(j, _):
        for k in range(D // L):
            xl_rows0[j, pl.ds(k * L, L)] = z16
        return 0

    lax.fori_loop(0, B, _zrow, 0)

    for g in range(B // L):
        w_buf0[pl.ds(g * L, L)] = z16
    row0 = sid * ROWS_PER_SUB
    for j in range(ROWS_PER_SUB // B):
        pltpu.sync_copy(xl_rows0, acc_sh.at[pl.ds(row0 + j * B, B)])
        pltpu.sync_copy(w_buf0, s_sh.at[pl.ds(row0 + j * B, B)])
    plsc.subcore_barrier()

    att_regs = [att_v[pl.ds(k * L, L)] for k in range(D // L)]

    def _issue_idx(t, ii):
        src_i, dst_i, isem = ibufs[ii]
        base = pl.multiple_of((wid + t * NW) * B, B)
        pltpu.make_async_copy(src_hbm.at[pl.ds(base, B)], src_i, isem).start()
        pltpu.make_async_copy(dst_hbm.at[pl.ds(base, B)], dst_i, isem).start()

    def _wait_idx(ii):
        src_i, dst_i, isem = ibufs[ii]
        pltpu.make_async_copy(src_hbm.at[pl.ds(0, B)], src_i, isem).wait()
        pltpu.make_async_copy(dst_hbm.at[pl.ds(0, B)], dst_i, isem).wait()

    def _wait_scatters(rb):
        xl_rows, xr_rows, w_buf, gsem, ssem = rbufs[rb]
        pltpu.make_async_copy(xl_rows, acc_sh.at[dst_i0], ssem).wait()
        pltpu.make_async_copy(w_buf, s_sh.at[dst_i0], ssem).wait()

    def _start_xr(rb, ii):
        xl_rows, xr_rows, w_buf, gsem, ssem = rbufs[rb]
        src_i, dst_i, isem = ibufs[ii]
        pltpu.make_async_copy(xr_hbm.at[dst_i], xr_rows, gsem).start()

    def _start_xl(rb, ii):
        xl_rows, xr_rows, w_buf, gsem, ssem = rbufs[rb]
        src_i, dst_i, isem = ibufs[ii]
        pltpu.make_async_copy(xl_hbm.at[src_i], xl_rows, gsem).start()

    def _section(t, j):
        p, q = j % 2, (j + 1) % 2
        xl_rows, xr_rows, w_buf, gsem, ssem = rbufs[p]
        src_i, dst_i, isem = ibufs[j]
        pltpu.make_async_copy(xl_hbm.at[src_i], xl_rows, gsem).wait()
        pltpu.make_async_copy(xr_hbm.at[dst_i], xr_rows, gsem).wait()

        def _edge_e(i, _):
            acc = jnp.zeros((L,), jnp.float32)
            for k in range(D // L):
                h = xl_rows[i, pl.ds(k * L, L)] + xr_rows[i, pl.ds(k * L, L)]
                h = jnp.maximum(h, h * 0.2)
                acc = acc + h * att_regs[k]
            e_stage[pl.ds(i * L, L)] = acc
            return 0

        lax.fori_loop(0, B, _edge_e, 0, unroll=4)

        @pl.when(t + 2 < MINCH)
        def _():
            _wait_idx((j + 2) % 4)
            _start_xr(p, (j + 2) % 4)

        @pl.when(t >= 1)
        def _():
            _wait_scatters(q)

        @pl.when(t + 1 < MINCH)
        def _():
            _start_xl(q, (j + 1) % 4)

        lane = lax.iota(jnp.int32, L)
        for g in range(B // L):
            fbase = lane * L + (g * L * L)
            ev = jnp.zeros((L,), jnp.float32)
            for k in range(L):
                ev = ev + plsc.load_gather(e_stage, [fbase + k])
            w_buf[pl.ds(g * L, L)] = jnp.exp(ev)

        def _edge_scale(i, _):
            wj = plsc.load_gather(w_buf, [jnp.full((L,), i, jnp.int32)])
            for k in range(D // L):
                sl = pl.ds(k * L, L)
                xl_rows[i, sl] = xl_rows[i, sl] * wj
            return 0

        lax.fori_loop(0, B, _edge_scale, 0, unroll=4)

        pltpu.async_copy(xl_rows, acc_sh.at[dst_i], ssem, add=True)
        pltpu.async_copy(w_buf, s_sh.at[dst_i], ssem, add=True)

        @pl.when(t + 3 < MINCH)
        def _():
            _issue_idx(t + 3, (j + 3) % 4)

    _issue_idx(0, 0)
    _issue_idx(1, 1)
    _issue_idx(2, 2)
    _wait_idx(0)
    _start_xl(0, 0)
    _start_xr(0, 0)
    _wait_idx(1)
    _start_xr(1, 1)

    def _quad(qq, _):
        for j in range(4):
            _section(4 * qq + j, j)
        return 0

    lax.fori_loop(0, MINCH // 4, _quad, 0)
    if MINCH % 4 == 1:
        _section(MINCH - 1, 0)
    else:
        raise NotImplementedError

    _wait_scatters(0)
    plsc.subcore_barrier()
    pltpu.sync_copy(acc_sh.at[pl.ds(row0, ROWS_PER_SUB)],
                    acc_out.at[cid, pl.ds(row0, ROWS_PER_SUB)])
    pltpu.sync_copy(s_sh.at[pl.ds(row0, ROWS_PER_SUB)],
                    s_out.at[cid, pl.ds(row0, ROWS_PER_SUB)])


def _fin_body(xl_ref, xr_ref, acc_ref, s_ref, att_ref, bias_ref, gamma_ref,
              beta_ref, out_ref):
    xl = xl_ref[...]
    xr = xr_ref[...]
    att = att_ref[...]
    h = xl + xr
    h = jnp.maximum(h, h * 0.2)
    e_self = jnp.sum(h * att, axis=1, keepdims=True)
    w_self = jnp.exp(e_self)
    num = acc_ref[0] + acc_ref[1] + w_self * xl
    den = jnp.sum(s_ref[...], axis=1, keepdims=True) + w_self
    out = num / den + bias_ref[...]
    out = jnp.maximum(out, out * 0.01)
    mu = jnp.mean(out, axis=1, keepdims=True)
    c = out - mu
    var = jnp.mean(c * c, axis=1, keepdims=True)
    out_ref[...] = c * lax.rsqrt(var + 1e-5) * gamma_ref[...] + beta_ref[...]


def _finish(xl, xr, acc2, s2t, att, bias, gamma, beta):
    blk = 2000
    grid = N // blk
    return pl.pallas_call(
        _fin_body,
        grid=(grid,),
        in_specs=[
            pl.BlockSpec((blk, D), lambda i: (i, 0)),
            pl.BlockSpec((blk, D), lambda i: (i, 0)),
            pl.BlockSpec((NC, blk, D), lambda i: (0, i, 0)),
            pl.BlockSpec((blk, NC), lambda i: (i, 0)),
            pl.BlockSpec((1, D), lambda i: (0, 0)),
            pl.BlockSpec((1, D), lambda i: (0, 0)),
            pl.BlockSpec((1, D), lambda i: (0, 0)),
            pl.BlockSpec((1, D), lambda i: (0, 0)),
        ],
        out_specs=pl.BlockSpec((blk, D), lambda i: (i, 0)),
        out_shape=jax.ShapeDtypeStruct((N, D), jnp.float32),
    )(xl, xr, acc2, s2t, att, bias, gamma, beta)


def kernel(x, edge_index, W_l, W_r, att, bias, gamma, beta):
    xl, xr = _matmuls(x, W_l, W_r)
    acc2, s2 = _edge_pass(xl, xr, edge_index[0], edge_index[1], att)
    s2t = s2.T
    return _finish(xl, xr, acc2, s2t,
                   att.reshape(1, D), bias.reshape(1, D),
                   gamma.reshape(1, D), beta.reshape(1, D))
